# baseline (speedup 1.0000x reference)
import functools

import jax
import jax.numpy as jnp
from jax import lax
from jax.experimental import pallas as pl
from jax.experimental.pallas import tpu as pltpu
from jax.experimental.pallas import tpu_sc as plsc

_NSC = 2
_NSUB = 16
_NW = _NSC * _NSUB
_K = 128
_BN = 1024


def _round_up(x, m):
    return (x + m - 1) // m * m


@functools.lru_cache(maxsize=None)
def _edge_agg(n_pad: int, e_pad: int, d: int):
    per_worker = e_pad // _NW
    n_chunks = per_worker // _K
    assert per_worker % _K == 0 and e_pad % _NW == 0
    rows_per_sub = n_pad // _NSUB
    assert n_pad % _NSUB == 0

    mesh = plsc.VectorSubcoreMesh(core_axis_name="c", subcore_axis_name="s")

    @functools.partial(
        pl.kernel,
        out_type=jax.ShapeDtypeStruct((_NSC * n_pad, d), jnp.float32),
        mesh=mesh,
        scratch_types=[
            pltpu.VMEM((n_chunks, _K), jnp.int32),
            pltpu.VMEM((n_chunks, _K), jnp.int32),
            pltpu.VMEM((_K, d), jnp.float32),
            pltpu.VMEM_SHARED((n_pad, d), jnp.float32),
            pltpu.SemaphoreType.DMA,
        ],
    )
    def agg_kernel(h_hbm, src_hbm, dst_hbm, z_hbm, out_hbm,
                   srcv, dstv, rows, slab, sem):
        c = lax.axis_index("c")
        s = lax.axis_index("s")
        wid = c * _NSUB + s
        r0 = s * rows_per_sub
        pltpu.sync_copy(z_hbm.at[pl.ds(r0, rows_per_sub)],
                        slab.at[pl.ds(r0, rows_per_sub)])
        pltpu.sync_copy(src_hbm.at[pl.ds(wid * n_chunks, n_chunks)], srcv)
        pltpu.sync_copy(dst_hbm.at[pl.ds(wid * n_chunks, n_chunks)], dstv)
        plsc.subcore_barrier()

        @pl.loop(0, n_chunks)
        def _(i):
            pltpu.async_copy(h_hbm.at[srcv.at[i]], rows, sem).wait()
            pltpu.sync_copy(rows, slab.at[dstv.at[i]], add=True)

        plsc.subcore_barrier()
        pltpu.sync_copy(slab.at[pl.ds(r0, rows_per_sub)],
                        out_hbm.at[pl.ds(c * n_pad + r0, rows_per_sub)])

    return agg_kernel


@functools.lru_cache(maxsize=None)
def _ones_agg(n_pad: int, e_pad: int, d: int):
    per_worker = e_pad // _NW
    n_chunks = per_worker // _K
    assert per_worker % _K == 0 and e_pad % _NW == 0
    rows_per_sub = n_pad // _NSUB
    assert n_pad % _NSUB == 0

    mesh = plsc.VectorSubcoreMesh(core_axis_name="c", subcore_axis_name="s")

    @functools.partial(
        pl.kernel,
        out_type=jax.ShapeDtypeStruct((_NSC * n_pad, d), jnp.float32),
        mesh=mesh,
        scratch_types=[
            pltpu.VMEM((n_chunks, _K), jnp.int32),
            pltpu.VMEM((_K, d), jnp.float32),
            pltpu.VMEM_SHARED((n_pad, d), jnp.float32),
        ],
    )
    def deg_kernel(idx_hbm, ones_hbm, z_hbm, out_hbm, idxv, ones, slab):
        c = lax.axis_index("c")
        s = lax.axis_index("s")
        wid = c * _NSUB + s
        r0 = s * rows_per_sub
        pltpu.sync_copy(z_hbm.at[pl.ds(r0, rows_per_sub)],
                        slab.at[pl.ds(r0, rows_per_sub)])
        pltpu.sync_copy(ones_hbm, ones)
        pltpu.sync_copy(idx_hbm.at[pl.ds(wid * n_chunks, n_chunks)], idxv)
        plsc.subcore_barrier()

        @pl.loop(0, n_chunks)
        def _(i):
            pltpu.sync_copy(ones, slab.at[idxv.at[i]], add=True)

        plsc.subcore_barrier()
        pltpu.sync_copy(slab.at[pl.ds(r0, rows_per_sub)],
                        out_hbm.at[pl.ds(c * n_pad + r0, rows_per_sub)])

    return deg_kernel


def _dis(degp_ref):
    dcnt = degp_ref[0] + degp_ref[1]
    return jnp.where(dcnt > 0.0,
                     1.0 / jnp.sqrt(jnp.maximum(dcnt, 1.0)), 0.0)[:, :1]


def _prep_body(feat_ref, dout_ref, o_ref):
    o_ref[...] = feat_ref[...] * _dis(dout_ref)


def _mm(x, w):
    return lax.dot_general(x, w, (((1,), (0,)), ((), ())),
                           precision=lax.Precision.HIGHEST,
                           preferred_element_type=jnp.float32)


def _l0_body(a_ref, din_ref, dout_ref, w_ref, b_ref, o_ref):
    agg = (a_ref[0] + a_ref[1]) * _dis(din_ref)
    h = jnp.maximum(_mm(agg, w_ref[...]) + b_ref[...], 0.0) * _dis(dout_ref)
    o_ref[0] = h[:, :128]
    o_ref[1] = h[:, 128:]


def _l1_body(aa_ref, ab_ref, din_ref, dout_ref, w1_ref, b1_ref, w2_ref, o_ref):
    din = _dis(din_ref)
    agg = jnp.concatenate([(aa_ref[0] + aa_ref[1]) * din,
                           (ab_ref[0] + ab_ref[1]) * din], axis=1)
    h = jnp.maximum(_mm(agg, w1_ref[...]) + b1_ref[...], 0.0) * _dis(dout_ref)
    o_ref[...] = _mm(h, w2_ref[...])


def _fin_body(a_ref, din_ref, b2_ref, o_ref):
    o_ref[...] = (a_ref[0] + a_ref[1]) * _dis(din_ref) + b2_ref[...]


def _row_block(d, rank3=False):
    if rank3:
        return pl.BlockSpec((2, _BN, d), lambda i: (0, i, 0))
    return pl.BlockSpec((_BN, d), lambda i: (i, 0))


def _full_block(shape):
    nd = len(shape)
    return pl.BlockSpec(shape, lambda i: (0,) * nd)


def _tc_call(body, n_pad, in_specs, out_dim, rank3_out=False):
    if rank3_out:
        out_shape = jax.ShapeDtypeStruct((2, n_pad, out_dim), jnp.float32)
        out_spec = _row_block(out_dim, rank3=True)
    else:
        out_shape = jax.ShapeDtypeStruct((n_pad, out_dim), jnp.float32)
        out_spec = _row_block(out_dim)
    return pl.pallas_call(
        body,
        grid=(n_pad // _BN,),
        in_specs=in_specs,
        out_specs=out_spec,
        out_shape=out_shape,
    )


def kernel(features, edge_index, W0, b0, W1, b1, W2, b2):
    n, in_dim = features.shape
    e = edge_index.shape[1]
    hid = W0.shape[1]
    ncls = W2.shape[1]
    n_pad = _round_up(n, _BN)
    assert n_pad % _NSUB == 0
    e_pad = _round_up(e, _NW * _K * 8)
    nc_pad = _round_up(ncls, 128)

    pad_e = e_pad - e
    src = jnp.concatenate([edge_index[0],
                           jnp.full((pad_e,), n_pad - 1, jnp.int32)])
    dst = jnp.concatenate([edge_index[1],
                           jnp.full((pad_e,), n_pad - 1, jnp.int32)])
    src = src.reshape(e_pad // _K, _K)
    dst = dst.reshape(e_pad // _K, _K)
    feat_p = jnp.pad(features, ((0, n_pad - n), (0, 0)))
    w2_p = jnp.pad(W2, ((0, 0), (0, nc_pad - ncls)))
    b0r = b0.reshape(1, hid)
    b1r = b1.reshape(1, hid)
    b2r = jnp.pad(b2, (0, nc_pad - ncls)).reshape(1, nc_pad)

    z128 = jnp.zeros((n_pad, 128), jnp.float32)
    ones_k = jnp.ones((_K, 128), jnp.float32)

    agg128 = _edge_agg(n_pad, e_pad, 128)
    deg128 = _ones_agg(n_pad, e_pad, 128)

    deg_out_p = deg128(src, ones_k, z128).reshape(2, n_pad, 128)
    deg_in_p = deg128(dst, ones_k, z128).reshape(2, n_pad, 128)

    dspec = _row_block(128, rank3=True)

    h0 = _tc_call(_prep_body, n_pad,
                  [_row_block(in_dim), dspec], in_dim)(feat_p, deg_out_p)

    a0 = agg128(h0, src, dst, z128).reshape(2, n_pad, 128)
    h1 = _tc_call(_l0_body, n_pad,
                  [_row_block(128, rank3=True), dspec, dspec,
                   _full_block((in_dim, hid)), _full_block((1, hid))],
                  128, rank3_out=True)(a0, deg_in_p, deg_out_p, W0, b0r)

    a1a = agg128(h1[0], src, dst, z128).reshape(2, n_pad, 128)
    a1b = agg128(h1[1], src, dst, z128).reshape(2, n_pad, 128)
    z = _tc_call(_l1_body, n_pad,
                 [_row_block(128, rank3=True), _row_block(128, rank3=True),
                  dspec, dspec, _full_block((hid, hid)), _full_block((1, hid)),
                  _full_block((hid, nc_pad))],
                 nc_pad)(a1a, a1b, deg_in_p, deg_out_p, W1, b1r, w2_p)

    a2 = agg128(z, src, dst, z128).reshape(2, n_pad, nc_pad)
    out = _tc_call(_fin_body, n_pad,
                   [_row_block(nc_pad, rank3=True), dspec,
                    _full_block((1, nc_pad))],
                   nc_pad)(a2, deg_in_p, b2r)

    return out[:n, :ncls]


# speedup vs baseline: 1.0880x; 1.0880x over previous
import functools

import jax
import jax.numpy as jnp
from jax import lax
from jax.experimental import pallas as pl
from jax.experimental.pallas import tpu as pltpu
from jax.experimental.pallas import tpu_sc as plsc

_NSC = 2
_NSUB = 16
_NW = _NSC * _NSUB
_K = 128
_BN = 1024


def _round_up(x, m):
    return (x + m - 1) // m * m


@functools.lru_cache(maxsize=None)
def _edge_agg(n_pad: int, e_pad: int, d: int, a_sub: int):
    total_chunks = e_pad // _K
    assert e_pad % _K == 0 and total_chunks % _NSUB == 0
    b_sub = total_chunks // _NSUB - a_sub
    assert a_sub > 0 and b_sub > 0
    rows_per_sub = n_pad // _NSUB
    assert n_pad % _NSUB == 0

    mesh = plsc.VectorSubcoreMesh(core_axis_name="c", subcore_axis_name="s")

    @functools.partial(
        pl.kernel,
        out_type=jax.ShapeDtypeStruct((_NSC * n_pad, d), jnp.float32),
        mesh=mesh,
        scratch_types=[
            pltpu.VMEM((_K,), jnp.int32),
            pltpu.VMEM((_K,), jnp.int32),
            pltpu.VMEM((_K, d), jnp.float32),
            pltpu.VMEM_SHARED((n_pad, d), jnp.float32),
            pltpu.SemaphoreType.DMA,
        ],
    )
    def agg_kernel(h_hbm, src_hbm, dst_hbm, z_hbm, out_hbm,
                   srcv, dstv, rows, slab, sem):
        c = lax.axis_index("c")
        s = lax.axis_index("s")
        r0 = s * rows_per_sub
        pltpu.sync_copy(z_hbm.at[pl.ds(r0, rows_per_sub)],
                        slab.at[pl.ds(r0, rows_per_sub)])
        plsc.subcore_barrier()

        def chunk_body(chunk):
            off = chunk * _K
            pltpu.sync_copy(src_hbm.at[pl.ds(off, _K)], srcv)
            pltpu.sync_copy(dst_hbm.at[pl.ds(off, _K)], dstv)
            pltpu.async_copy(h_hbm.at[srcv], rows, sem).wait()
            pltpu.sync_copy(rows, slab.at[dstv], add=True)

        @pl.when(c == 0)
        def _():
            @pl.loop(0, a_sub)
            def _(i):
                chunk_body(s * a_sub + i)

        @pl.when(c == 1)
        def _():
            @pl.loop(0, b_sub)
            def _(i):
                chunk_body(_NSUB * a_sub + s * b_sub + i)

        plsc.subcore_barrier()
        pltpu.sync_copy(slab.at[pl.ds(r0, rows_per_sub)],
                        out_hbm.at[pl.ds(c * n_pad + r0, rows_per_sub)])

    return agg_kernel


@functools.lru_cache(maxsize=None)
def _ones_agg(n_pad: int, e_pad: int, d: int):
    per_worker = e_pad // _NW
    n_chunks = per_worker // _K
    assert per_worker % _K == 0 and e_pad % _NW == 0
    rows_per_sub = n_pad // _NSUB
    assert n_pad % _NSUB == 0

    mesh = plsc.VectorSubcoreMesh(core_axis_name="c", subcore_axis_name="s")

    @functools.partial(
        pl.kernel,
        out_type=jax.ShapeDtypeStruct((_NSC * n_pad, d), jnp.float32),
        mesh=mesh,
        scratch_types=[
            pltpu.VMEM((_K,), jnp.int32),
            pltpu.VMEM((_K, d), jnp.float32),
            pltpu.VMEM_SHARED((n_pad, d), jnp.float32),
        ],
    )
    def deg_kernel(idx_hbm, ones_hbm, z_hbm, out_hbm, idxv, ones, slab):
        c = lax.axis_index("c")
        s = lax.axis_index("s")
        wid = c * _NSUB + s
        r0 = s * rows_per_sub
        pltpu.sync_copy(z_hbm.at[pl.ds(r0, rows_per_sub)],
                        slab.at[pl.ds(r0, rows_per_sub)])
        pltpu.sync_copy(ones_hbm, ones)
        plsc.subcore_barrier()
        base = wid * per_worker

        @pl.loop(0, n_chunks)
        def _(i):
            off = base + i * _K
            pltpu.sync_copy(idx_hbm.at[pl.ds(off, _K)], idxv)
            pltpu.sync_copy(ones, slab.at[idxv], add=True)

        plsc.subcore_barrier()
        pltpu.sync_copy(slab.at[pl.ds(r0, rows_per_sub)],
                        out_hbm.at[pl.ds(c * n_pad + r0, rows_per_sub)])

    return deg_kernel


def _dis(degp_ref):
    dcnt = degp_ref[0] + degp_ref[1]
    return jnp.where(dcnt > 0.0,
                     1.0 / jnp.sqrt(jnp.maximum(dcnt, 1.0)), 0.0)[:, :1]


def _prep_body(feat_ref, dout_ref, o_ref):
    o_ref[...] = feat_ref[...] * _dis(dout_ref)


def _mm(x, w):
    return lax.dot_general(x, w, (((1,), (0,)), ((), ())),
                           precision=lax.Precision.HIGHEST,
                           preferred_element_type=jnp.float32)


def _l0_body(a_ref, din_ref, dout_ref, w_ref, b_ref, o_ref):
    agg = (a_ref[0] + a_ref[1]) * _dis(din_ref)
    h = jnp.maximum(_mm(agg, w_ref[...]) + b_ref[...], 0.0) * _dis(dout_ref)
    o_ref[0] = h[:, :128]
    o_ref[1] = h[:, 128:]


def _l1_body(aa_ref, ab_ref, din_ref, dout_ref, w1_ref, b1_ref, w2_ref, o_ref):
    din = _dis(din_ref)
    agg = jnp.concatenate([(aa_ref[0] + aa_ref[1]) * din,
                           (ab_ref[0] + ab_ref[1]) * din], axis=1)
    h = jnp.maximum(_mm(agg, w1_ref[...]) + b1_ref[...], 0.0) * _dis(dout_ref)
    o_ref[...] = _mm(h, w2_ref[...])


def _fin_body(a_ref, din_ref, b2_ref, o_ref):
    o_ref[...] = (a_ref[0] + a_ref[1]) * _dis(din_ref) + b2_ref[...]


def _row_block(d, rank3=False):
    if rank3:
        return pl.BlockSpec((2, _BN, d), lambda i: (0, i, 0))
    return pl.BlockSpec((_BN, d), lambda i: (i, 0))


def _full_block(shape):
    nd = len(shape)
    return pl.BlockSpec(shape, lambda i: (0,) * nd)


def _tc_call(body, n_pad, in_specs, out_dim, rank3_out=False):
    if rank3_out:
        out_shape = jax.ShapeDtypeStruct((2, n_pad, out_dim), jnp.float32)
        out_spec = _row_block(out_dim, rank3=True)
    else:
        out_shape = jax.ShapeDtypeStruct((n_pad, out_dim), jnp.float32)
        out_spec = _row_block(out_dim)
    return pl.pallas_call(
        body,
        grid=(n_pad // _BN,),
        in_specs=in_specs,
        out_specs=out_spec,
        out_shape=out_shape,
    )


def kernel(features, edge_index, W0, b0, W1, b1, W2, b2):
    n, in_dim = features.shape
    e = edge_index.shape[1]
    hid = W0.shape[1]
    ncls = W2.shape[1]
    n_pad = _round_up(n, _BN)
    assert n_pad % _NSUB == 0
    e_pad = _round_up(e, _NW * _K)
    nc_pad = _round_up(ncls, 128)

    pad_e = e_pad - e
    src = jnp.concatenate([edge_index[0],
                           jnp.full((pad_e,), n_pad - 1, jnp.int32)])
    dst = jnp.concatenate([edge_index[1],
                           jnp.full((pad_e,), n_pad - 1, jnp.int32)])
    feat_p = jnp.pad(features, ((0, n_pad - n), (0, 0)))
    w2_p = jnp.pad(W2, ((0, 0), (0, nc_pad - ncls)))
    b0r = b0.reshape(1, hid)
    b1r = b1.reshape(1, hid)
    b2r = jnp.pad(b2, (0, nc_pad - ncls)).reshape(1, nc_pad)

    z128 = jnp.zeros((n_pad, 128), jnp.float32)
    ones_k = jnp.ones((_K, 128), jnp.float32)

    total_sub = e_pad // _K // _NSUB
    a_sub = max(1, (total_sub * 3) // 8)
    agg128 = _edge_agg(n_pad, e_pad, 128, a_sub)
    deg128 = _ones_agg(n_pad, e_pad, 128)

    deg_out_p = deg128(src, ones_k, z128).reshape(2, n_pad, 128)
    deg_in_p = deg128(dst, ones_k, z128).reshape(2, n_pad, 128)

    dspec = _row_block(128, rank3=True)

    h0 = _tc_call(_prep_body, n_pad,
                  [_row_block(in_dim), dspec], in_dim)(feat_p, deg_out_p)

    a0 = agg128(h0, src, dst, z128).reshape(2, n_pad, 128)
    h1 = _tc_call(_l0_body, n_pad,
                  [_row_block(128, rank3=True), dspec, dspec,
                   _full_block((in_dim, hid)), _full_block((1, hid))],
                  128, rank3_out=True)(a0, deg_in_p, deg_out_p, W0, b0r)

    a1a = agg128(h1[0], src, dst, z128).reshape(2, n_pad, 128)
    a1b = agg128(h1[1], src, dst, z128).reshape(2, n_pad, 128)
    z = _tc_call(_l1_body, n_pad,
                 [_row_block(128, rank3=True), _row_block(128, rank3=True),
                  dspec, dspec, _full_block((hid, hid)), _full_block((1, hid)),
                  _full_block((hid, nc_pad))],
                 nc_pad)(a1a, a1b, deg_in_p, deg_out_p, W1, b1r, w2_p)

    a2 = agg128(z, src, dst, z128).reshape(2, n_pad, nc_pad)
    out = _tc_call(_fin_body, n_pad,
                   [_row_block(nc_pad, rank3=True), dspec,
                    _full_block((1, nc_pad))],
                   nc_pad)(a2, deg_in_p, b2r)

    return out[:n, :ncls]


# speedup vs baseline: 1.3374x; 1.2292x over previous
import functools

import jax
import jax.numpy as jnp
from jax import lax
from jax.experimental import pallas as pl
from jax.experimental.pallas import tpu as pltpu
from jax.experimental.pallas import tpu_sc as plsc

_NSC = 2
_NSUB = 16
_NW = _NSC * _NSUB
_K = 128
_BN = 1024


def _round_up(x, m):
    return (x + m - 1) // m * m


@functools.lru_cache(maxsize=None)
def _edge_agg(n_pad: int, e_pad: int, d: int, a_sub: int):
    total_chunks = e_pad // _K
    assert e_pad % _K == 0 and total_chunks % _NSUB == 0
    b_sub = total_chunks // _NSUB - a_sub
    assert a_sub > 0 and b_sub > 0
    rows_per_sub = n_pad // _NSUB
    assert n_pad % _NSUB == 0

    mesh = plsc.VectorSubcoreMesh(core_axis_name="c", subcore_axis_name="s")

    @functools.partial(
        pl.kernel,
        out_type=jax.ShapeDtypeStruct((_NSC * n_pad, d), jnp.float32),
        mesh=mesh,
        scratch_types=[
            pltpu.VMEM((_K,), jnp.int32),
            pltpu.VMEM((_K,), jnp.int32),
            pltpu.VMEM((_K, d), jnp.float32),
            pltpu.VMEM_SHARED((n_pad, d), jnp.float32),
            pltpu.SemaphoreType.DMA,
        ],
    )
    def agg_kernel(h_hbm, src_hbm, dst_hbm, z_hbm, out_hbm,
                   srcv, dstv, rows, slab, sem):
        c = lax.axis_index("c")
        s = lax.axis_index("s")
        r0 = s * rows_per_sub
        pltpu.sync_copy(z_hbm.at[pl.ds(r0, rows_per_sub)],
                        slab.at[pl.ds(r0, rows_per_sub)])
        plsc.subcore_barrier()

        def chunk_body(chunk):
            off = chunk * _K
            pltpu.sync_copy(src_hbm.at[pl.ds(off, _K)], srcv)
            pltpu.sync_copy(dst_hbm.at[pl.ds(off, _K)], dstv)
            pltpu.async_copy(h_hbm.at[srcv], rows, sem).wait()
            pltpu.sync_copy(rows, slab.at[dstv], add=True)

        @pl.when(c == 0)
        def _():
            @pl.loop(0, a_sub)
            def _(i):
                chunk_body(s * a_sub + i)

        @pl.when(c == 1)
        def _():
            @pl.loop(0, b_sub)
            def _(i):
                chunk_body(_NSUB * a_sub + s * b_sub + i)

        plsc.subcore_barrier()
        pltpu.sync_copy(slab.at[pl.ds(r0, rows_per_sub)],
                        out_hbm.at[pl.ds(c * n_pad + r0, rows_per_sub)])

    return agg_kernel


@functools.lru_cache(maxsize=None)
def _ones_agg(n_pad: int, e_pad: int, d: int):
    per_worker = e_pad // _NW
    n_chunks = per_worker // _K
    assert per_worker % _K == 0 and e_pad % _NW == 0
    rows_per_sub = n_pad // _NSUB
    assert n_pad % _NSUB == 0

    mesh = plsc.VectorSubcoreMesh(core_axis_name="c", subcore_axis_name="s")

    @functools.partial(
        pl.kernel,
        out_type=jax.ShapeDtypeStruct((_NSC * n_pad, d), jnp.float32),
        mesh=mesh,
        scratch_types=[
            pltpu.VMEM((_K,), jnp.int32),
            pltpu.VMEM((_K, d), jnp.float32),
            pltpu.VMEM_SHARED((n_pad, d), jnp.float32),
        ],
    )
    def deg_kernel(idx_hbm, ones_hbm, z_hbm, out_hbm, idxv, ones, slab):
        c = lax.axis_index("c")
        s = lax.axis_index("s")
        wid = c * _NSUB + s
        r0 = s * rows_per_sub
        pltpu.sync_copy(z_hbm.at[pl.ds(r0, rows_per_sub)],
                        slab.at[pl.ds(r0, rows_per_sub)])
        pltpu.sync_copy(ones_hbm, ones)
        plsc.subcore_barrier()
        base = wid * per_worker

        @pl.loop(0, n_chunks)
        def _(i):
            off = base + i * _K
            pltpu.sync_copy(idx_hbm.at[pl.ds(off, _K)], idxv)
            pltpu.sync_copy(ones, slab.at[idxv], add=True)

        plsc.subcore_barrier()
        pltpu.sync_copy(slab.at[pl.ds(r0, rows_per_sub)],
                        out_hbm.at[pl.ds(c * n_pad + r0, rows_per_sub)])

    return deg_kernel


def _dis(degp_ref):
    dcnt = degp_ref[0] + degp_ref[1]
    return jnp.where(dcnt > 0.0,
                     1.0 / jnp.sqrt(jnp.maximum(dcnt, 1.0)), 0.0)[:, :1]


def _prep_body(feat_ref, dout_ref, o_ref):
    o_ref[...] = feat_ref[...] * _dis(dout_ref)


def _mm(x, w):
    return lax.dot_general(x, w, (((1,), (0,)), ((), ())),
                           precision=lax.Precision.HIGHEST,
                           preferred_element_type=jnp.float32)


def _l0_body(a_ref, din_ref, dout_ref, w_ref, b_ref, o_ref):
    agg = (a_ref[0] + a_ref[1]) * _dis(din_ref)
    h = jnp.maximum(_mm(agg, w_ref[...]) + b_ref[...], 0.0) * _dis(dout_ref)
    o_ref[0] = h[:, :128]
    o_ref[1] = h[:, 128:]


def _l1_body(aa_ref, ab_ref, din_ref, dout_ref, w1_ref, b1_ref, w2_ref, o_ref):
    din = _dis(din_ref)
    agg = jnp.concatenate([(aa_ref[0] + aa_ref[1]) * din,
                           (ab_ref[0] + ab_ref[1]) * din], axis=1)
    h = jnp.maximum(_mm(agg, w1_ref[...]) + b1_ref[...], 0.0) * _dis(dout_ref)
    o_ref[...] = _mm(h, w2_ref[...])


def _fin_body(a_ref, din_ref, b2_ref, o_ref):
    o_ref[...] = (a_ref[0] + a_ref[1]) * _dis(din_ref) + b2_ref[...]


def _row_block(d, rank3=False):
    if rank3:
        return pl.BlockSpec((2, _BN, d), lambda i: (0, i, 0))
    return pl.BlockSpec((_BN, d), lambda i: (i, 0))


def _full_block(shape):
    nd = len(shape)
    return pl.BlockSpec(shape, lambda i: (0,) * nd)


def _tc_call(body, n_pad, in_specs, out_dim, rank3_out=False):
    if rank3_out:
        out_shape = jax.ShapeDtypeStruct((2, n_pad, out_dim), jnp.float32)
        out_spec = _row_block(out_dim, rank3=True)
    else:
        out_shape = jax.ShapeDtypeStruct((n_pad, out_dim), jnp.float32)
        out_spec = _row_block(out_dim)
    return pl.pallas_call(
        body,
        grid=(n_pad // _BN,),
        in_specs=in_specs,
        out_specs=out_spec,
        out_shape=out_shape,
    )


def kernel(features, edge_index, W0, b0, W1, b1, W2, b2):
    n, in_dim = features.shape
    e = edge_index.shape[1]
    hid = W0.shape[1]
    ncls = W2.shape[1]
    n_pad = _round_up(n, _BN)
    assert n_pad % _NSUB == 0
    e_pad = _round_up(e, _NW * _K)
    nc_pad = _round_up(ncls, 128)

    pad_e = e_pad - e
    src = jnp.concatenate([edge_index[0],
                           jnp.full((pad_e,), n_pad - 1, jnp.int32)])
    dst = jnp.concatenate([edge_index[1],
                           jnp.full((pad_e,), n_pad - 1, jnp.int32)])
    feat_p = jnp.pad(features, ((0, n_pad - n), (0, 0)))
    w2_p = jnp.pad(W2, ((0, 0), (0, nc_pad - ncls)))
    b0r = b0.reshape(1, hid)
    b1r = b1.reshape(1, hid)
    b2r = jnp.pad(b2, (0, nc_pad - ncls)).reshape(1, nc_pad)

    z128 = jnp.zeros((n_pad, 128), jnp.float32)
    ones_k = jnp.ones((_K, 128), jnp.float32)

    total_sub = e_pad // _K // _NSUB
    a_sub = max(1, (total_sub * 5) // 8)
    agg128 = _edge_agg(n_pad, e_pad, 128, a_sub)
    deg128 = _ones_agg(n_pad, e_pad, 128)

    deg_out_p = deg128(src, ones_k, z128).reshape(2, n_pad, 128)
    deg_in_p = deg128(dst, ones_k, z128).reshape(2, n_pad, 128)

    dspec = _row_block(128, rank3=True)

    h0 = _tc_call(_prep_body, n_pad,
                  [_row_block(in_dim), dspec], in_dim)(feat_p, deg_out_p)

    a0 = agg128(h0, src, dst, z128).reshape(2, n_pad, 128)
    h1 = _tc_call(_l0_body, n_pad,
                  [_row_block(128, rank3=True), dspec, dspec,
                   _full_block((in_dim, hid)), _full_block((1, hid))],
                  128, rank3_out=True)(a0, deg_in_p, deg_out_p, W0, b0r)

    a1a = agg128(h1[0], src, dst, z128).reshape(2, n_pad, 128)
    a1b = agg128(h1[1], src, dst, z128).reshape(2, n_pad, 128)
    z = _tc_call(_l1_body, n_pad,
                 [_row_block(128, rank3=True), _row_block(128, rank3=True),
                  dspec, dspec, _full_block((hid, hid)), _full_block((1, hid)),
                  _full_block((hid, nc_pad))],
                 nc_pad)(a1a, a1b, deg_in_p, deg_out_p, W1, b1r, w2_p)

    a2 = agg128(z, src, dst, z128).reshape(2, n_pad, nc_pad)
    out = _tc_call(_fin_body, n_pad,
                   [_row_block(nc_pad, rank3=True), dspec,
                    _full_block((1, nc_pad))],
                   nc_pad)(a2, deg_in_p, b2r)

    return out[:n, :ncls]


# speedup vs baseline: 1.4205x; 1.0622x over previous
import functools

import jax
import jax.numpy as jnp
from jax import lax
from jax.experimental import pallas as pl
from jax.experimental.pallas import tpu as pltpu
from jax.experimental.pallas import tpu_sc as plsc

_NSC = 2
_NSUB = 16
_NW = _NSC * _NSUB
_K = 128
_BN = 1024


def _round_up(x, m):
    return (x + m - 1) // m * m


@functools.lru_cache(maxsize=None)
def _edge_agg(n_pad: int, e_pad: int, d: int, a_sub: int):
    total_chunks = e_pad // _K
    assert e_pad % _K == 0 and total_chunks % _NSUB == 0
    b_sub = total_chunks // _NSUB - a_sub
    assert a_sub > 0 and b_sub > 0
    rows_per_sub = n_pad // _NSUB
    assert n_pad % _NSUB == 0

    mesh = plsc.VectorSubcoreMesh(core_axis_name="c", subcore_axis_name="s")

    @functools.partial(
        pl.kernel,
        out_type=jax.ShapeDtypeStruct((_NSC * n_pad, d), jnp.float32),
        mesh=mesh,
        scratch_types=[
            pltpu.VMEM((_K,), jnp.int32),
            pltpu.VMEM((_K,), jnp.int32),
            pltpu.VMEM((_K, d), jnp.float32),
            pltpu.VMEM_SHARED((n_pad, d), jnp.float32),
            pltpu.SemaphoreType.DMA,
        ],
    )
    def agg_kernel(h_hbm, src_hbm, dst_hbm, z_hbm, out_hbm,
                   srcv, dstv, rows, slab, sem):
        c = lax.axis_index("c")
        s = lax.axis_index("s")
        r0 = s * rows_per_sub
        pltpu.sync_copy(z_hbm.at[pl.ds(r0, rows_per_sub)],
                        slab.at[pl.ds(r0, rows_per_sub)])
        plsc.subcore_barrier()

        def chunk_body(chunk):
            off = chunk * _K
            pltpu.sync_copy(src_hbm.at[pl.ds(off, _K)], srcv)
            pltpu.sync_copy(dst_hbm.at[pl.ds(off, _K)], dstv)
            pltpu.async_copy(h_hbm.at[srcv], rows, sem).wait()
            pltpu.sync_copy(rows, slab.at[dstv], add=True)

        @pl.when(c == 0)
        def _():
            @pl.loop(0, a_sub)
            def _(i):
                chunk_body(s * a_sub + i)

        @pl.when(c == 1)
        def _():
            @pl.loop(0, b_sub)
            def _(i):
                chunk_body(_NSUB * a_sub + s * b_sub + i)

        plsc.subcore_barrier()
        pltpu.sync_copy(slab.at[pl.ds(r0, rows_per_sub)],
                        out_hbm.at[pl.ds(c * n_pad + r0, rows_per_sub)])

    return agg_kernel


@functools.lru_cache(maxsize=None)
def _ones_agg(n_pad: int, e_pad: int, d: int):
    per_worker = e_pad // _NW
    n_chunks = per_worker // _K
    assert per_worker % _K == 0 and e_pad % _NW == 0
    rows_per_sub = n_pad // _NSUB
    assert n_pad % _NSUB == 0

    mesh = plsc.VectorSubcoreMesh(core_axis_name="c", subcore_axis_name="s")

    @functools.partial(
        pl.kernel,
        out_type=jax.ShapeDtypeStruct((_NSC * n_pad, d), jnp.float32),
        mesh=mesh,
        scratch_types=[
            pltpu.VMEM((_K,), jnp.int32),
            pltpu.VMEM((_K, d), jnp.float32),
            pltpu.VMEM_SHARED((n_pad, d), jnp.float32),
        ],
    )
    def deg_kernel(idx_hbm, ones_hbm, z_hbm, out_hbm, idxv, ones, slab):
        c = lax.axis_index("c")
        s = lax.axis_index("s")
        wid = c * _NSUB + s
        r0 = s * rows_per_sub
        pltpu.sync_copy(z_hbm.at[pl.ds(r0, rows_per_sub)],
                        slab.at[pl.ds(r0, rows_per_sub)])
        pltpu.sync_copy(ones_hbm, ones)
        plsc.subcore_barrier()
        base = wid * per_worker

        @pl.loop(0, n_chunks)
        def _(i):
            off = base + i * _K
            pltpu.sync_copy(idx_hbm.at[pl.ds(off, _K)], idxv)
            pltpu.sync_copy(ones, slab.at[idxv], add=True)

        plsc.subcore_barrier()
        pltpu.sync_copy(slab.at[pl.ds(r0, rows_per_sub)],
                        out_hbm.at[pl.ds(c * n_pad + r0, rows_per_sub)])

    return deg_kernel


def _dis(degp_ref):
    dcnt = degp_ref[0] + degp_ref[1]
    return jnp.where(dcnt > 0.0,
                     1.0 / jnp.sqrt(jnp.maximum(dcnt, 1.0)), 0.0)[:, :1]


def _prep_body(feat_ref, dout_ref, o_ref):
    o_ref[...] = feat_ref[...] * _dis(dout_ref)


def _mm(x, w):
    return lax.dot_general(x, w, (((1,), (0,)), ((), ())),
                           precision=lax.Precision.HIGHEST,
                           preferred_element_type=jnp.float32)


def _l0_body(a_ref, din_ref, dout_ref, w_ref, b_ref, o_ref):
    agg = (a_ref[0] + a_ref[1]) * _dis(din_ref)
    h = jnp.maximum(_mm(agg, w_ref[...]) + b_ref[...], 0.0) * _dis(dout_ref)
    o_ref[0] = h[:, :128]
    o_ref[1] = h[:, 128:]


def _l1_body(aa_ref, ab_ref, din_ref, dout_ref, w1_ref, b1_ref, w2_ref, o_ref):
    din = _dis(din_ref)
    agg = jnp.concatenate([(aa_ref[0] + aa_ref[1]) * din,
                           (ab_ref[0] + ab_ref[1]) * din], axis=1)
    h = jnp.maximum(_mm(agg, w1_ref[...]) + b1_ref[...], 0.0) * _dis(dout_ref)
    o_ref[...] = _mm(h, w2_ref[...])


def _fin_body(a_ref, din_ref, b2_ref, o_ref):
    o_ref[...] = (a_ref[0] + a_ref[1]) * _dis(din_ref) + b2_ref[...]


def _row_block(d, rank3=False):
    if rank3:
        return pl.BlockSpec((2, _BN, d), lambda i: (0, i, 0))
    return pl.BlockSpec((_BN, d), lambda i: (i, 0))


def _full_block(shape):
    nd = len(shape)
    return pl.BlockSpec(shape, lambda i: (0,) * nd)


def _tc_call(body, n_pad, in_specs, out_dim, rank3_out=False):
    if rank3_out:
        out_shape = jax.ShapeDtypeStruct((2, n_pad, out_dim), jnp.float32)
        out_spec = _row_block(out_dim, rank3=True)
    else:
        out_shape = jax.ShapeDtypeStruct((n_pad, out_dim), jnp.float32)
        out_spec = _row_block(out_dim)
    return pl.pallas_call(
        body,
        grid=(n_pad // _BN,),
        in_specs=in_specs,
        out_specs=out_spec,
        out_shape=out_shape,
    )


def kernel(features, edge_index, W0, b0, W1, b1, W2, b2):
    n, in_dim = features.shape
    e = edge_index.shape[1]
    hid = W0.shape[1]
    ncls = W2.shape[1]
    n_pad = _round_up(n, _BN)
    assert n_pad % _NSUB == 0
    e_pad = _round_up(e, _NW * _K)
    nc_pad = _round_up(ncls, 128)

    pad_e = e_pad - e
    src = jnp.concatenate([edge_index[0],
                           jnp.full((pad_e,), n_pad - 1, jnp.int32)])
    dst = jnp.concatenate([edge_index[1],
                           jnp.full((pad_e,), n_pad - 1, jnp.int32)])
    feat_p = jnp.pad(features, ((0, n_pad - n), (0, 0)))
    w2_p = jnp.pad(W2, ((0, 0), (0, nc_pad - ncls)))
    b0r = b0.reshape(1, hid)
    b1r = b1.reshape(1, hid)
    b2r = jnp.pad(b2, (0, nc_pad - ncls)).reshape(1, nc_pad)

    z128 = jnp.zeros((n_pad, 128), jnp.float32)
    dw = 32
    ones_k = jnp.ones((_K, dw), jnp.float32)
    z_dw = jnp.zeros((n_pad, dw), jnp.float32)

    total_sub = e_pad // _K // _NSUB
    a_sub = max(1, (total_sub * 21) // 32)
    agg128 = _edge_agg(n_pad, e_pad, 128, a_sub)
    degk = _ones_agg(n_pad, e_pad, dw)

    deg_out_p = degk(src, ones_k, z_dw).reshape(2, n_pad, dw)
    deg_in_p = degk(dst, ones_k, z_dw).reshape(2, n_pad, dw)

    dspec = _row_block(dw, rank3=True)

    h0 = _tc_call(_prep_body, n_pad,
                  [_row_block(in_dim), dspec], in_dim)(feat_p, deg_out_p)

    a0 = agg128(h0, src, dst, z128).reshape(2, n_pad, 128)
    h1 = _tc_call(_l0_body, n_pad,
                  [_row_block(128, rank3=True), dspec, dspec,
                   _full_block((in_dim, hid)), _full_block((1, hid))],
                  128, rank3_out=True)(a0, deg_in_p, deg_out_p, W0, b0r)

    a1a = agg128(h1[0], src, dst, z128).reshape(2, n_pad, 128)
    a1b = agg128(h1[1], src, dst, z128).reshape(2, n_pad, 128)
    z = _tc_call(_l1_body, n_pad,
                 [_row_block(128, rank3=True), _row_block(128, rank3=True),
                  dspec, dspec, _full_block((hid, hid)), _full_block((1, hid)),
                  _full_block((hid, nc_pad))],
                 nc_pad)(a1a, a1b, deg_in_p, deg_out_p, W1, b1r, w2_p)

    a2 = agg128(z, src, dst, z128).reshape(2, n_pad, nc_pad)
    out = _tc_call(_fin_body, n_pad,
                   [_row_block(nc_pad, rank3=True), dspec,
                    _full_block((1, nc_pad))],
                   nc_pad)(a2, deg_in_p, b2r)

    return out[:n, :ncls]
